# Initial kernel scaffold; baseline (speedup 1.0000x reference)
import jax
import jax.numpy as jnp
from jax.experimental import pallas as pl


def kernel(node_feats, edge_feats, edge_index, W1, b1, W2, b2):
    raise NotImplementedError("write your pallas kernel here")


# speedup vs baseline: 3.1076x; 3.1076x over previous
import functools
import math

import jax
import jax.numpy as jnp
from jax import lax
from jax.experimental import pallas as pl
from jax.experimental.pallas import tpu as pltpu
from jax.experimental.pallas import tpu_sc as plsc

_NC = 2
_NS = 16
_NW = _NC * _NS

_CH = 80


def _sc_gather(table, idx):
    B = idx.shape[0]
    D = table.shape[1]
    b_per_w = B // _NW
    n_ch = b_per_w // _CH
    idx3d = idx.reshape(_NW, n_ch, _CH)

    mesh = plsc.VectorSubcoreMesh(core_axis_name="c", subcore_axis_name="s")

    @functools.partial(
        pl.kernel,
        out_type=jax.ShapeDtypeStruct((B, D), table.dtype),
        mesh=mesh,
        scratch_types=[
            pltpu.VMEM((n_ch, _CH), jnp.int32),
            pltpu.VMEM((_CH, D), table.dtype),
            pltpu.SemaphoreType.DMA,
        ],
    )
    def k(table_hbm, idx_hbm, out_hbm, idx_v, rows_v, gsem):
        wid = lax.axis_index("s") * _NC + lax.axis_index("c")
        pltpu.sync_copy(idx_hbm.at[wid], idx_v)

        def body(c, carry):
            pltpu.async_copy(table_hbm.at[idx_v.at[c]], rows_v, gsem).wait()
            pltpu.sync_copy(
                rows_v, out_hbm.at[pl.ds(wid * b_per_w + c * _CH, _CH)]
            )
            return carry

        lax.fori_loop(0, n_ch, body, 0)

    return k(table, idx3d)


def _mlp_body(ni_ref, nj_ref, ef_ref, w1_ref, b1_ref, w2_ref, b2_ref, out_ref):
    x = jnp.concatenate([ni_ref[...], nj_ref[...], ef_ref[...]], axis=1)
    h = lax.dot_general(
        x, w1_ref[...], (((1,), (1,)), ((), ())),
        preferred_element_type=jnp.float32,
    ) + b1_ref[...]
    h = 0.5 * h * (1.0 + lax.erf(h * (1.0 / math.sqrt(2.0))))
    out_ref[...] = lax.dot_general(
        h, w2_ref[...], (((1,), (1,)), ((), ())),
        preferred_element_type=jnp.float32,
    ) + b2_ref[...]


def _tc_mlp(gathered, edge_feats, W1, b1, W2, b2, n_edges, e_blk):
    n_blk = n_edges // e_blk
    node_dim = gathered.shape[1]
    edge_dim = edge_feats.shape[1]
    in_dim = W1.shape[1]
    out_dim = W2.shape[0]
    return pl.pallas_call(
        _mlp_body,
        grid=(n_blk,),
        in_specs=[
            pl.BlockSpec((e_blk, node_dim), lambda e: (e, 0)),
            pl.BlockSpec((e_blk, node_dim), lambda e: (e + n_blk, 0)),
            pl.BlockSpec((e_blk, edge_dim), lambda e: (e, 0)),
            pl.BlockSpec((in_dim, in_dim), lambda e: (0, 0)),
            pl.BlockSpec((1, in_dim), lambda e: (0, 0)),
            pl.BlockSpec((out_dim, in_dim), lambda e: (0, 0)),
            pl.BlockSpec((1, out_dim), lambda e: (0, 0)),
        ],
        out_specs=pl.BlockSpec((e_blk, out_dim), lambda e: (e, 0)),
        out_shape=jax.ShapeDtypeStruct((n_edges, out_dim), jnp.float32),
    )(gathered, gathered, edge_feats, W1, b1, W2, b2)


def kernel(node_feats, edge_feats, edge_index, W1, b1, W2, b2):
    n_edges = edge_feats.shape[0]
    idx_all = edge_index.reshape(-1).astype(jnp.int32)
    gathered = _sc_gather(node_feats, idx_all)
    return _tc_mlp(
        gathered, edge_feats, W1, b1[None, :], W2, b2[None, :],
        n_edges, e_blk=2560,
    )
